# Initial kernel scaffold; baseline (speedup 1.0000x reference)
import jax
import jax.numpy as jnp
from jax.experimental import pallas as pl

_NUM_TASKS = 512
_MAX_COUNT_CAP = 64
_LAG_WEEKS = 4
_CONFIDENCE_LOGIT = 12.0
_OFF_LOGIT = -12.0

_B_BLK = 32


def _onehot_kernel(lag_ref, out_ref):
    counts = jnp.clip(jnp.round(lag_ref[...]), 0.0, float(_MAX_COUNT_CAP))
    k = jax.lax.broadcasted_iota(
        jnp.float32, (_B_BLK, _NUM_TASKS, _MAX_COUNT_CAP + 1), 2
    )
    out_ref[...] = jnp.where(
        counts[:, :, None] == k, _CONFIDENCE_LOGIT, _OFF_LOGIT
    ).astype(out_ref.dtype)


@jax.jit
def kernel(sequence):
    batch_size, window_size, _ = sequence.shape
    lag = sequence[:, window_size - _LAG_WEEKS, :_NUM_TASKS]
    grid = (batch_size // _B_BLK,)
    return pl.pallas_call(
        _onehot_kernel,
        grid=grid,
        in_specs=[
            pl.BlockSpec((_B_BLK, _NUM_TASKS), lambda i: (i, 0)),
        ],
        out_specs=pl.BlockSpec(
            (_B_BLK, _NUM_TASKS, _MAX_COUNT_CAP + 1), lambda i: (i, 0, 0)
        ),
        out_shape=jax.ShapeDtypeStruct(
            (batch_size, _NUM_TASKS, _MAX_COUNT_CAP + 1), sequence.dtype
        ),
    )(lag)


# speedup vs baseline: 8.4551x; 8.4551x over previous
import jax
import jax.numpy as jnp
from jax.experimental import pallas as pl

_NUM_TASKS = 512
_MAX_COUNT_CAP = 64
_LAG_WEEKS = 4
_CONFIDENCE_LOGIT = 12.0
_OFF_LOGIT = -12.0

_B_BLK = 32


def _onehot_kernel(lag_ref, out_ref):
    counts = jnp.clip(
        jnp.round(lag_ref[...]).astype(jnp.int32), 0, _MAX_COUNT_CAP
    )
    k = jax.lax.broadcasted_iota(
        jnp.int32, (_B_BLK, _NUM_TASKS, _MAX_COUNT_CAP + 1), 2
    )
    out_ref[...] = jnp.where(
        counts[:, :, None] == k, _CONFIDENCE_LOGIT, _OFF_LOGIT
    ).astype(out_ref.dtype)


@jax.jit
def kernel(sequence):
    batch_size, window_size, _ = sequence.shape
    lag = sequence[:, window_size - _LAG_WEEKS, :_NUM_TASKS]
    grid = (batch_size // _B_BLK,)
    return pl.pallas_call(
        _onehot_kernel,
        grid=grid,
        in_specs=[
            pl.BlockSpec((_B_BLK, _NUM_TASKS), lambda i: (i, 0)),
        ],
        out_specs=pl.BlockSpec(
            (_B_BLK, _NUM_TASKS, _MAX_COUNT_CAP + 1), lambda i: (i, 0, 0)
        ),
        out_shape=jax.ShapeDtypeStruct(
            (batch_size, _NUM_TASKS, _MAX_COUNT_CAP + 1), sequence.dtype
        ),
    )(lag)
